# baseline (speedup 1.0000x reference)
import functools

import jax
import jax.numpy as jnp
from jax import lax
from jax.experimental import pallas as pl
from jax.experimental.pallas import tpu as pltpu
from jax.experimental.pallas import tpu_sc as plsc

N = 10000
E = 320000
D = 128
G = 20
GP = 32
MAXN = 500
NP = 10240
NC = 2
NS = 16
NW = NC * NS
CH = 64
EP = 327680
NCHUNK = EP // CH
CPW = EP // NW // CH
RPT = NP // NS
F32 = jnp.float32
_PH = jax.lax.Precision.HIGHEST

NBUF = 5
LA = 2
IB = 8
LI = 3


def _dot(a, b):
    return jnp.dot(a, b, preferred_element_type=F32)


def _dotx(a, b):
    return jnp.dot(a, b, precision=_PH, preferred_element_type=F32)


def _rd(a):
    return a.astype(jnp.bfloat16).astype(F32)


def _make_sc_agg(compute_deg: bool):
    mesh = plsc.VectorSubcoreMesh(core_axis_name="c", subcore_axis_name="s",
                                  num_cores=NC, num_subcores=NS)
    out_type = [jax.ShapeDtypeStruct((NC, NP, D), F32)]
    if compute_deg:
        out_type.append(jax.ShapeDtypeStruct((NC, NP), F32))
    scratch = [pltpu.VMEM((2, CH), jnp.int32) for _ in range(IB)]
    scratch += [pltpu.VMEM((CH, D), F32) for _ in range(NBUF)]
    scratch += [
        pltpu.VMEM((CH,), F32),
        pltpu.VMEM((CH,), F32),
    ]
    scratch += [pltpu.SemaphoreType.DMA] * (IB + 3 * NBUF)
    scratch += [
        pltpu.VMEM_SHARED((NP, D), F32),
        pltpu.VMEM_SHARED((NP,), F32),
    ]

    def body(h_hbm, sd_hbm, zrows_hbm, zdeg_hbm, ones_hbm, *refs):
        if compute_deg:
            agg_out, deg_out = refs[0], refs[1]
            rest = refs[2:]
        else:
            agg_out = refs[0]
            deg_out = None
            rest = refs[1:]
        islots = list(rest[0:IB])
        bufs = list(rest[IB:IB + NBUF])
        ones_v, zv = rest[IB + NBUF], rest[IB + NBUF + 1]
        sems = list(rest[IB + NBUF + 2:-2])
        agg_sh, deg_sh = rest[-2], rest[-1]
        isem = sems[0:IB]
        gsem = sems[IB:IB + NBUF]
        ssem = sems[IB + NBUF:IB + 2 * NBUF]
        dsem = sems[IB + 2 * NBUF:IB + 3 * NBUF]
        c = lax.axis_index("c")
        s = lax.axis_index("s")
        wid = s * NC + c
        base = wid * CPW

        pltpu.sync_copy(zrows_hbm, bufs[0])
        for k in range(RPT // CH):
            pltpu.sync_copy(bufs[0], agg_sh.at[pl.ds(s * RPT + k * CH, CH)])
        pltpu.sync_copy(zdeg_hbm, zv)
        if compute_deg:
            for k in range(RPT // CH):
                pltpu.sync_copy(zv, deg_sh.at[pl.ds(s * RPT + k * CH, CH)])
            pltpu.sync_copy(ones_hbm, ones_v)
        plsc.subcore_barrier()

        def idx_load(u):
            pltpu.async_copy(sd_hbm.at[base + u], islots[u % IB],
                             isem[u % IB])

        def gather(t, b):
            return pltpu.async_copy(h_hbm.at[islots[t % IB].at[0]], bufs[b],
                                    gsem[b])

        for k in range(LI):
            idx_load(k)
        for k in range(LA):
            pltpu.make_async_copy(sd_hbm.at[base + k], islots[k % IB],
                                  isem[k % IB]).wait()
            gather(k, k % NBUF)

        for t in range(CPW):
            b = t % NBUF
            ti = t + LI
            if ti < CPW:
                idx_load(ti)
            tf = t + LA
            if tf < CPW:
                bf = tf % NBUF
                if tf >= NBUF:
                    pltpu.make_async_copy(
                        bufs[bf], agg_sh.at[islots[(tf - NBUF) % IB].at[1]],
                        ssem[bf]).wait()
                pltpu.make_async_copy(sd_hbm.at[base + tf],
                                      islots[tf % IB], isem[tf % IB]).wait()
                gather(tf, bf)
            pltpu.make_async_copy(h_hbm.at[islots[t % IB].at[0]], bufs[b],
                                  gsem[b]).wait()
            pltpu.async_copy(bufs[b], agg_sh.at[islots[t % IB].at[1]],
                             ssem[b], add=True)
            if compute_deg:
                if t >= NBUF:
                    pltpu.make_async_copy(
                        ones_v, deg_sh.at[islots[(t - NBUF) % IB].at[1]],
                        dsem[b]).wait()
                pltpu.async_copy(ones_v, deg_sh.at[islots[t % IB].at[1]],
                                 dsem[b], add=True)

        for t in range(CPW - NBUF, CPW):
            b = t % NBUF
            pltpu.make_async_copy(bufs[b], agg_sh.at[islots[t % IB].at[1]],
                                  ssem[b]).wait()
            if compute_deg:
                pltpu.make_async_copy(ones_v,
                                      deg_sh.at[islots[t % IB].at[1]],
                                      dsem[b]).wait()
        plsc.subcore_barrier()

        for k in range(RPT // CH):
            off = s * RPT + k * CH
            b = k % NBUF
            pltpu.sync_copy(agg_sh.at[pl.ds(off, CH)], bufs[b])
            pltpu.sync_copy(bufs[b], agg_out.at[c, pl.ds(off, CH)])
            if compute_deg:
                pltpu.sync_copy(deg_sh.at[pl.ds(off, CH)], zv)
                pltpu.sync_copy(zv, deg_out.at[c, pl.ds(off, CH)])

    return pl.kernel(body, out_type=tuple(out_type), mesh=mesh,
                     scratch_types=scratch)


_sc_agg_deg = _make_sc_agg(True)
_sc_agg = _make_sc_agg(False)


BLK = 1024
NBLK = NP // BLK
INF = 16


def _tc_input(xp, w_t, b):
    def body(x_ref, w_ref, b_ref, o_ref):
        o_ref[...] = jnp.maximum(_dot(x_ref[...], w_ref[...]) + b_ref[...],
                                 0.0)
    return pl.pallas_call(
        body,
        grid=(NBLK,),
        in_specs=[
            pl.BlockSpec((BLK, INF), lambda i: (i, 0)),
            pl.BlockSpec((INF, D), lambda i: (0, 0)),
            pl.BlockSpec((1, D), lambda i: (0, 0)),
        ],
        out_specs=pl.BlockSpec((BLK, D), lambda i: (i, 0)),
        out_shape=jax.ShapeDtypeStruct((NP, D), F32))(xp, w_t, b)


def _tc_layer(aggp, degp3, h, wl_t, bl, wr_t):
    def body(a_ref, dg_ref, h_ref, wl_ref, bl_ref, wr_ref, o_ref):
        agg = a_ref[0] + a_ref[1]
        deg = dg_ref[0] + dg_ref[1]
        deginv = 1.0 / jnp.maximum(deg, 1.0)
        m = _dot(agg * deginv, wl_ref[...])
        r = _dot(h_ref[...], wr_ref[...])
        o_ref[...] = jnp.maximum(m + bl_ref[...] + r, 0.0)
    return pl.pallas_call(
        body,
        grid=(NBLK,),
        in_specs=[
            pl.BlockSpec((NC, BLK, D), lambda i: (0, i, 0)),
            pl.BlockSpec((NC, BLK, 1), lambda i: (0, i, 0)),
            pl.BlockSpec((BLK, D), lambda i: (i, 0)),
            pl.BlockSpec((D, D), lambda i: (0, 0)),
            pl.BlockSpec((1, D), lambda i: (0, 0)),
            pl.BlockSpec((D, D), lambda i: (0, 0)),
        ],
        out_specs=pl.BlockSpec((BLK, D), lambda i: (i, 0)),
        out_shape=jax.ShapeDtypeStruct((NP, D), F32))(
            aggp, degp3, h, wl_t, bl, wr_t)


def _tc_final(h, batchp, wa_t, ba, s_row, wq_t, bq, wv_t, bv, wo_t, bo,
              wb_t, bb):
    def body(h_ref, b_ref, wa_ref, ba_ref, s_ref, wq_ref, bq_ref, wv_ref,
             bv_ref, wo_ref, bo_ref, wb_ref, bb_ref, o_ref):
        gid = lax.broadcasted_iota(jnp.int32, (BLK, GP), 1)
        counts = jnp.zeros((1, GP), F32)
        for i in range(NBLK):
            bat = b_ref[pl.ds(i * BLK, BLK), :]
            mask = (bat == gid).astype(F32)
            counts = counts + jnp.sum(mask, axis=0, keepdims=True)
        r1 = lax.broadcasted_iota(jnp.int32, (GP, GP), 0)
        c1 = lax.broadcasted_iota(jnp.int32, (GP, GP), 1)
        ut = (r1 < c1).astype(F32)
        starts = _dotx(counts, ut)
        starts_col = starts.reshape(GP, 1)
        row0 = lax.broadcasted_iota(jnp.int32, (BLK, 1), 0).astype(F32)
        gsum = jnp.zeros((GP, D), F32)
        for i in range(NBLK):
            bat = b_ref[pl.ds(i * BLK, BLK), :]
            mask = (bat == gid).astype(F32)
            out_b = _dot(h_ref[pl.ds(i * BLK, BLK), :], wa_ref[...]) \
                + ba_ref[...]
            starts_pn = _dotx(mask, starts_col)
            pos = row0 + float(i * BLK) - starts_pn
            validf = jnp.where(pos < float(MAXN), 1.0, 0.0)
            gsum = gsum + lax.dot_general(
                mask, _rd(out_b) * validf, (((0,), (0,)), ((), ())),
                precision=_PH, preferred_element_type=F32)
        q = _dot(s_ref[...], wq_ref[...]) + bq_ref[...]
        o = q + _dotx(gsum, _rd(wv_ref[...])) \
            + float(MAXN) * bv_ref[...]
        o = o + jnp.maximum(_dot(o, wo_ref[...]) + bo_ref[...], 0.0)
        o_ref[...] = _dot(o, wb_ref[...]) + bb_ref[...]
    return pl.pallas_call(
        body, out_shape=jax.ShapeDtypeStruct((GP, 1), F32))(
            h, batchp, wa_t, ba, s_row, wq_t, bq, wv_t, bv, wo_t, bo,
            wb_t, bb)


def kernel(x, edge_index, batch, params):
    p = params
    epad = jnp.full((EP - E,), NP - 1, jnp.int32)
    src = jnp.concatenate([edge_index[0].astype(jnp.int32), epad])
    dst = jnp.concatenate([edge_index[1].astype(jnp.int32), epad])
    sd = jnp.stack([src.reshape(NCHUNK, CH), dst.reshape(NCHUNK, CH)],
                   axis=1)
    xp = jnp.pad(x.astype(F32), ((0, NP - N), (0, 0)))
    batchp = jnp.pad(batch.astype(jnp.int32), (0, NP - N),
                     constant_values=GP - 1).reshape(NP, 1)
    zrows = jnp.zeros((CH, D), F32)
    zdeg = jnp.zeros((CH,), F32)
    ones_ch = jnp.ones((CH,), F32)

    h = _tc_input(xp, p['W_in'].T, p['b_in'].reshape(1, D))

    aggp, degp = _sc_agg_deg(h, sd, zrows, zdeg, ones_ch)
    degp3 = degp.reshape(NC, NP, 1)
    h = _tc_layer(aggp, degp3, h, p['W_l0'].T, p['b_l0'].reshape(1, D),
                  p['W_r0'].T)
    for l in (1, 2):
        aggp = _sc_agg(h, sd, zrows, zdeg, ones_ch)[0]
        h = _tc_layer(aggp, degp3, h, p['W_l%d' % l].T,
                      p['b_l%d' % l].reshape(1, D), p['W_r%d' % l].T)

    res = _tc_final(h, batchp, p['W_a'].T, p['b_a'].reshape(1, D),
                    p['S'].reshape(1, D), p['W_q'].T, p['b_q'].reshape(1, D),
                    p['W_v'].T, p['b_v'].reshape(1, D), p['W_o'].T,
                    p['b_o'].reshape(1, D), p['W_b'].T,
                    p['b_b'].reshape(1, 1))
    return res[:G, 0]


# speedup vs baseline: 1.1274x; 1.1274x over previous
import functools

import jax
import jax.numpy as jnp
from jax import lax
from jax.experimental import pallas as pl
from jax.experimental.pallas import tpu as pltpu
from jax.experimental.pallas import tpu_sc as plsc

N = 10000
E = 320000
D = 128
G = 20
GP = 32
MAXN = 500
NP = 10240
NC = 2
NS = 16
NW = NC * NS
CH = 128
EP = 327680
NCHUNK = EP // CH
CPW = EP // NW // CH
RPT = NP // NS
F32 = jnp.float32
_PH = jax.lax.Precision.HIGHEST

NBUF = 2
LA = 1
IB = 4
LI = 2


def _dot(a, b):
    return jnp.dot(a, b, preferred_element_type=F32)


def _dotx(a, b):
    return jnp.dot(a, b, precision=_PH, preferred_element_type=F32)


def _rd(a):
    return a.astype(jnp.bfloat16).astype(F32)


def _make_sc_agg(compute_deg: bool):
    mesh = plsc.VectorSubcoreMesh(core_axis_name="c", subcore_axis_name="s",
                                  num_cores=NC, num_subcores=NS)
    out_type = [jax.ShapeDtypeStruct((NC, NP, D), F32)]
    if compute_deg:
        out_type.append(jax.ShapeDtypeStruct((NC, NP), F32))
    scratch = [pltpu.VMEM((2, CH), jnp.int32) for _ in range(IB)]
    scratch += [pltpu.VMEM((CH, D), F32) for _ in range(NBUF)]
    scratch += [
        pltpu.VMEM((CH,), F32),
        pltpu.VMEM((CH,), F32),
    ]
    scratch += [pltpu.SemaphoreType.DMA] * (IB + 3 * NBUF)
    scratch += [
        pltpu.VMEM_SHARED((NP, D), F32),
        pltpu.VMEM_SHARED((NP,), F32),
    ]

    def body(h_hbm, sd_hbm, zrows_hbm, zdeg_hbm, ones_hbm, *refs):
        if compute_deg:
            agg_out, deg_out = refs[0], refs[1]
            rest = refs[2:]
        else:
            agg_out = refs[0]
            deg_out = None
            rest = refs[1:]
        islots = list(rest[0:IB])
        bufs = list(rest[IB:IB + NBUF])
        ones_v, zv = rest[IB + NBUF], rest[IB + NBUF + 1]
        sems = list(rest[IB + NBUF + 2:-2])
        agg_sh, deg_sh = rest[-2], rest[-1]
        isem = sems[0:IB]
        gsem = sems[IB:IB + NBUF]
        ssem = sems[IB + NBUF:IB + 2 * NBUF]
        dsem = sems[IB + 2 * NBUF:IB + 3 * NBUF]
        c = lax.axis_index("c")
        s = lax.axis_index("s")
        wid = s * NC + c
        base = wid * CPW

        pltpu.sync_copy(zrows_hbm, bufs[0])
        for k in range(RPT // CH):
            pltpu.sync_copy(bufs[0], agg_sh.at[pl.ds(s * RPT + k * CH, CH)])
        pltpu.sync_copy(zdeg_hbm, zv)
        if compute_deg:
            for k in range(RPT // CH):
                pltpu.sync_copy(zv, deg_sh.at[pl.ds(s * RPT + k * CH, CH)])
            pltpu.sync_copy(ones_hbm, ones_v)
        plsc.subcore_barrier()

        def idx_load(u):
            pltpu.async_copy(sd_hbm.at[base + u], islots[u % IB],
                             isem[u % IB])

        def gather(t, b):
            return pltpu.async_copy(h_hbm.at[islots[t % IB].at[0]], bufs[b],
                                    gsem[b])

        for k in range(LI):
            idx_load(k)
        for k in range(LA):
            pltpu.make_async_copy(sd_hbm.at[base + k], islots[k % IB],
                                  isem[k % IB]).wait()
            gather(k, k % NBUF)

        for t in range(CPW):
            b = t % NBUF
            ti = t + LI
            if ti < CPW:
                idx_load(ti)
            tf = t + LA
            if tf < CPW:
                bf = tf % NBUF
                if tf >= NBUF:
                    pltpu.make_async_copy(
                        bufs[bf], agg_sh.at[islots[(tf - NBUF) % IB].at[1]],
                        ssem[bf]).wait()
                pltpu.make_async_copy(sd_hbm.at[base + tf],
                                      islots[tf % IB], isem[tf % IB]).wait()
                gather(tf, bf)
            pltpu.make_async_copy(h_hbm.at[islots[t % IB].at[0]], bufs[b],
                                  gsem[b]).wait()
            pltpu.async_copy(bufs[b], agg_sh.at[islots[t % IB].at[1]],
                             ssem[b], add=True)
            if compute_deg:
                if t >= NBUF:
                    pltpu.make_async_copy(
                        ones_v, deg_sh.at[islots[(t - NBUF) % IB].at[1]],
                        dsem[b]).wait()
                pltpu.async_copy(ones_v, deg_sh.at[islots[t % IB].at[1]],
                                 dsem[b], add=True)

        for t in range(CPW - NBUF, CPW):
            b = t % NBUF
            pltpu.make_async_copy(bufs[b], agg_sh.at[islots[t % IB].at[1]],
                                  ssem[b]).wait()
            if compute_deg:
                pltpu.make_async_copy(ones_v,
                                      deg_sh.at[islots[t % IB].at[1]],
                                      dsem[b]).wait()
        plsc.subcore_barrier()

        for k in range(RPT // CH):
            off = s * RPT + k * CH
            b = k % NBUF
            pltpu.sync_copy(agg_sh.at[pl.ds(off, CH)], bufs[b])
            pltpu.sync_copy(bufs[b], agg_out.at[c, pl.ds(off, CH)])
            if compute_deg:
                pltpu.sync_copy(deg_sh.at[pl.ds(off, CH)], zv)
                pltpu.sync_copy(zv, deg_out.at[c, pl.ds(off, CH)])

    return pl.kernel(body, out_type=tuple(out_type), mesh=mesh,
                     scratch_types=scratch)


_sc_agg_deg = _make_sc_agg(True)
_sc_agg = _make_sc_agg(False)


BLK = 1024
NBLK = NP // BLK
INF = 16


def _tc_input(xp, w_t, b):
    def body(x_ref, w_ref, b_ref, o_ref):
        o_ref[...] = jnp.maximum(_dot(x_ref[...], w_ref[...]) + b_ref[...],
                                 0.0)
    return pl.pallas_call(
        body,
        grid=(NBLK,),
        in_specs=[
            pl.BlockSpec((BLK, INF), lambda i: (i, 0)),
            pl.BlockSpec((INF, D), lambda i: (0, 0)),
            pl.BlockSpec((1, D), lambda i: (0, 0)),
        ],
        out_specs=pl.BlockSpec((BLK, D), lambda i: (i, 0)),
        out_shape=jax.ShapeDtypeStruct((NP, D), F32))(xp, w_t, b)


def _tc_layer(aggp, degp3, h, wl_t, bl, wr_t):
    def body(a_ref, dg_ref, h_ref, wl_ref, bl_ref, wr_ref, o_ref):
        agg = a_ref[0] + a_ref[1]
        deg = dg_ref[0] + dg_ref[1]
        deginv = 1.0 / jnp.maximum(deg, 1.0)
        m = _dot(agg * deginv, wl_ref[...])
        r = _dot(h_ref[...], wr_ref[...])
        o_ref[...] = jnp.maximum(m + bl_ref[...] + r, 0.0)
    return pl.pallas_call(
        body,
        grid=(NBLK,),
        in_specs=[
            pl.BlockSpec((NC, BLK, D), lambda i: (0, i, 0)),
            pl.BlockSpec((NC, BLK, 1), lambda i: (0, i, 0)),
            pl.BlockSpec((BLK, D), lambda i: (i, 0)),
            pl.BlockSpec((D, D), lambda i: (0, 0)),
            pl.BlockSpec((1, D), lambda i: (0, 0)),
            pl.BlockSpec((D, D), lambda i: (0, 0)),
        ],
        out_specs=pl.BlockSpec((BLK, D), lambda i: (i, 0)),
        out_shape=jax.ShapeDtypeStruct((NP, D), F32))(
            aggp, degp3, h, wl_t, bl, wr_t)


def _tc_final(h, batchp, wa_t, ba, s_row, wq_t, bq, wv_t, bv, wo_t, bo,
              wb_t, bb):
    def body(h_ref, b_ref, wa_ref, ba_ref, s_ref, wq_ref, bq_ref, wv_ref,
             bv_ref, wo_ref, bo_ref, wb_ref, bb_ref, o_ref):
        gid = lax.broadcasted_iota(jnp.int32, (BLK, GP), 1)
        counts = jnp.zeros((1, GP), F32)
        for i in range(NBLK):
            bat = b_ref[pl.ds(i * BLK, BLK), :]
            mask = (bat == gid).astype(F32)
            counts = counts + jnp.sum(mask, axis=0, keepdims=True)
        r1 = lax.broadcasted_iota(jnp.int32, (GP, GP), 0)
        c1 = lax.broadcasted_iota(jnp.int32, (GP, GP), 1)
        ut = (r1 < c1).astype(F32)
        starts = _dotx(counts, ut)
        starts_col = starts.reshape(GP, 1)
        row0 = lax.broadcasted_iota(jnp.int32, (BLK, 1), 0).astype(F32)
        gsum = jnp.zeros((GP, D), F32)
        for i in range(NBLK):
            bat = b_ref[pl.ds(i * BLK, BLK), :]
            mask = (bat == gid).astype(F32)
            out_b = _dot(h_ref[pl.ds(i * BLK, BLK), :], wa_ref[...]) \
                + ba_ref[...]
            starts_pn = _dotx(mask, starts_col)
            pos = row0 + float(i * BLK) - starts_pn
            validf = jnp.where(pos < float(MAXN), 1.0, 0.0)
            gsum = gsum + lax.dot_general(
                mask, _rd(out_b) * validf, (((0,), (0,)), ((), ())),
                precision=_PH, preferred_element_type=F32)
        q = _dot(s_ref[...], wq_ref[...]) + bq_ref[...]
        o = q + _dotx(gsum, _rd(wv_ref[...])) \
            + float(MAXN) * bv_ref[...]
        o = o + jnp.maximum(_dot(o, wo_ref[...]) + bo_ref[...], 0.0)
        o_ref[...] = _dot(o, wb_ref[...]) + bb_ref[...]
    return pl.pallas_call(
        body, out_shape=jax.ShapeDtypeStruct((GP, 1), F32))(
            h, batchp, wa_t, ba, s_row, wq_t, bq, wv_t, bv, wo_t, bo,
            wb_t, bb)


def kernel(x, edge_index, batch, params):
    p = params
    epad = jnp.full((EP - E,), NP - 1, jnp.int32)
    src = jnp.concatenate([edge_index[0].astype(jnp.int32), epad])
    dst = jnp.concatenate([edge_index[1].astype(jnp.int32), epad])
    sd = jnp.stack([src.reshape(NCHUNK, CH), dst.reshape(NCHUNK, CH)],
                   axis=1)
    xp = jnp.pad(x.astype(F32), ((0, NP - N), (0, 0)))
    batchp = jnp.pad(batch.astype(jnp.int32), (0, NP - N),
                     constant_values=GP - 1).reshape(NP, 1)
    zrows = jnp.zeros((CH, D), F32)
    zdeg = jnp.zeros((CH,), F32)
    ones_ch = jnp.ones((CH,), F32)

    h = _tc_input(xp, p['W_in'].T, p['b_in'].reshape(1, D))

    aggp, degp = _sc_agg_deg(h, sd, zrows, zdeg, ones_ch)
    degp3 = degp.reshape(NC, NP, 1)
    h = _tc_layer(aggp, degp3, h, p['W_l0'].T, p['b_l0'].reshape(1, D),
                  p['W_r0'].T)
    for l in (1, 2):
        aggp = _sc_agg(h, sd, zrows, zdeg, ones_ch)[0]
        h = _tc_layer(aggp, degp3, h, p['W_l%d' % l].T,
                      p['b_l%d' % l].reshape(1, D), p['W_r%d' % l].T)

    res = _tc_final(h, batchp, p['W_a'].T, p['b_a'].reshape(1, D),
                    p['S'].reshape(1, D), p['W_q'].T, p['b_q'].reshape(1, D),
                    p['W_v'].T, p['b_v'].reshape(1, D), p['W_o'].T,
                    p['b_o'].reshape(1, D), p['W_b'].T,
                    p['b_b'].reshape(1, 1))
    return res[:G, 0]


# speedup vs baseline: 3.7701x; 3.3441x over previous
import functools

import jax
import jax.numpy as jnp
from jax import lax
from jax.experimental import pallas as pl
from jax.experimental.pallas import tpu as pltpu
from jax.experimental.pallas import tpu_sc as plsc

N = 10000
E = 320000
D = 128
G = 20
GP = 32
MAXN = 500
NP = 10240
NC = 2
NS = 16
NW = NC * NS
CH = 128
EP = 327680
NCHUNK = EP // CH
CPW = EP // NW // CH
RPT = NP // NS
F32 = jnp.float32
_PH = jax.lax.Precision.HIGHEST

NBUF = 2
LA = 1
IB = 4
LI = 2


def _dot(a, b):
    return jnp.dot(a, b, preferred_element_type=F32)


def _dotx(a, b):
    return jnp.dot(a, b, precision=_PH, preferred_element_type=F32)


def _rd(a):
    return a.astype(jnp.bfloat16).astype(F32)


def _make_sc_agg(compute_deg: bool):
    mesh = plsc.VectorSubcoreMesh(core_axis_name="c", subcore_axis_name="s",
                                  num_cores=NC, num_subcores=NS)
    out_type = [jax.ShapeDtypeStruct((NC, NP, D), F32)]
    if compute_deg:
        out_type.append(jax.ShapeDtypeStruct((NC, NP), F32))
    scratch = [pltpu.VMEM((2, CH), jnp.int32) for _ in range(IB)]
    scratch += [pltpu.VMEM((CH, D), F32) for _ in range(NBUF)]
    scratch += [
        pltpu.VMEM((CH,), F32),
        pltpu.VMEM((CH,), F32),
    ]
    scratch += [pltpu.SemaphoreType.DMA] * (IB + 3 * NBUF)
    scratch += [
        pltpu.VMEM_SHARED((NP, D), F32),
        pltpu.VMEM_SHARED((NP,), F32),
    ]

    def body(h_hbm, sd_hbm, zrows_hbm, zdeg_hbm, ones_hbm, *refs):
        if compute_deg:
            agg_out, deg_out = refs[0], refs[1]
            rest = refs[2:]
        else:
            agg_out = refs[0]
            deg_out = None
            rest = refs[1:]
        islots = list(rest[0:IB])
        bufs = list(rest[IB:IB + NBUF])
        ones_v, zv = rest[IB + NBUF], rest[IB + NBUF + 1]
        sems = list(rest[IB + NBUF + 2:-2])
        agg_sh, deg_sh = rest[-2], rest[-1]
        isem = sems[0:IB]
        gsem = sems[IB:IB + NBUF]
        ssem = sems[IB + NBUF:IB + 2 * NBUF]
        dsem = sems[IB + 2 * NBUF:IB + 3 * NBUF]
        c = lax.axis_index("c")
        s = lax.axis_index("s")
        wid = s * NC + c
        base = wid * CPW

        pltpu.sync_copy(zrows_hbm, bufs[0])
        for k in range(RPT // CH):
            pltpu.sync_copy(bufs[0], agg_sh.at[pl.ds(s * RPT + k * CH, CH)])
        pltpu.sync_copy(zdeg_hbm, zv)
        if compute_deg:
            for k in range(RPT // CH):
                pltpu.sync_copy(zv, deg_sh.at[pl.ds(s * RPT + k * CH, CH)])
            pltpu.sync_copy(ones_hbm, ones_v)
        plsc.subcore_barrier()

        def idx_load(u):
            pltpu.async_copy(sd_hbm.at[base + u], islots[u % IB],
                             isem[u % IB])

        def gather(t, b):
            return pltpu.async_copy(h_hbm.at[islots[t % IB].at[0]], bufs[b],
                                    gsem[b])

        for k in range(LI):
            idx_load(k)
        for k in range(LA):
            pltpu.make_async_copy(sd_hbm.at[base + k], islots[k % IB],
                                  isem[k % IB]).wait()
            gather(k, k % NBUF)

        for t in range(CPW):
            b = t % NBUF
            ti = t + LI
            if ti < CPW:
                idx_load(ti)
            tf = t + LA
            if tf < CPW:
                bf = tf % NBUF
                if tf >= NBUF:
                    pltpu.make_async_copy(
                        bufs[bf], agg_sh.at[islots[(tf - NBUF) % IB].at[1]],
                        ssem[bf]).wait()
                pltpu.make_async_copy(sd_hbm.at[base + tf],
                                      islots[tf % IB], isem[tf % IB]).wait()
                gather(tf, bf)
            pltpu.make_async_copy(h_hbm.at[islots[t % IB].at[0]], bufs[b],
                                  gsem[b]).wait()
            pltpu.async_copy(bufs[b], agg_sh.at[islots[t % IB].at[1]],
                             ssem[b], add=True)
            if compute_deg:
                if t >= NBUF:
                    pltpu.make_async_copy(
                        ones_v, deg_sh.at[islots[(t - NBUF) % IB].at[1]],
                        dsem[b]).wait()
                pltpu.async_copy(ones_v, deg_sh.at[islots[t % IB].at[1]],
                                 dsem[b], add=True)

        for t in range(CPW - NBUF, CPW):
            b = t % NBUF
            pltpu.make_async_copy(bufs[b], agg_sh.at[islots[t % IB].at[1]],
                                  ssem[b]).wait()
            if compute_deg:
                pltpu.make_async_copy(ones_v,
                                      deg_sh.at[islots[t % IB].at[1]],
                                      dsem[b]).wait()
        plsc.subcore_barrier()

        for k in range(RPT // CH):
            off = s * RPT + k * CH
            b = k % NBUF
            pltpu.sync_copy(agg_sh.at[pl.ds(off, CH)], bufs[b])
            pltpu.sync_copy(bufs[b], agg_out.at[c, pl.ds(off, CH)])
            if compute_deg:
                pltpu.sync_copy(deg_sh.at[pl.ds(off, CH)], zv)
                pltpu.sync_copy(zv, deg_out.at[c, pl.ds(off, CH)])

    return pl.kernel(body, out_type=tuple(out_type), mesh=mesh,
                     scratch_types=scratch)


_sc_agg_deg = _make_sc_agg(True)
_sc_agg = _make_sc_agg(False)


BLK = 1024
NBLK = NP // BLK
INF = 16


def _tc_input(xp, w_t, b):
    def body(x_ref, w_ref, b_ref, o_ref):
        o_ref[...] = jnp.maximum(_dot(x_ref[...], w_ref[...]) + b_ref[...],
                                 0.0)
    return pl.pallas_call(
        body,
        grid=(NBLK,),
        in_specs=[
            pl.BlockSpec((BLK, INF), lambda i: (i, 0)),
            pl.BlockSpec((INF, D), lambda i: (0, 0)),
            pl.BlockSpec((1, D), lambda i: (0, 0)),
        ],
        out_specs=pl.BlockSpec((BLK, D), lambda i: (i, 0)),
        out_shape=jax.ShapeDtypeStruct((NP, D), F32))(xp, w_t, b)


def _tc_layer(aggp, degp3, h, wl_t, bl, wr_t):
    def body(a_ref, dg_ref, h_ref, wl_ref, bl_ref, wr_ref, o_ref):
        agg = a_ref[0] + a_ref[1]
        deg = dg_ref[0] + dg_ref[1]
        deginv = 1.0 / jnp.maximum(deg, 1.0)
        m = _dot(agg * deginv, wl_ref[...])
        r = _dot(h_ref[...], wr_ref[...])
        o_ref[...] = jnp.maximum(m + bl_ref[...] + r, 0.0)
    return pl.pallas_call(
        body,
        grid=(NBLK,),
        in_specs=[
            pl.BlockSpec((NC, BLK, D), lambda i: (0, i, 0)),
            pl.BlockSpec((NC, BLK, 1), lambda i: (0, i, 0)),
            pl.BlockSpec((BLK, D), lambda i: (i, 0)),
            pl.BlockSpec((D, D), lambda i: (0, 0)),
            pl.BlockSpec((1, D), lambda i: (0, 0)),
            pl.BlockSpec((D, D), lambda i: (0, 0)),
        ],
        out_specs=pl.BlockSpec((BLK, D), lambda i: (i, 0)),
        out_shape=jax.ShapeDtypeStruct((NP, D), F32))(
            aggp, degp3, h, wl_t, bl, wr_t)


def _tc_final(h, batchp, wa_t, ba, s_row, wq_t, bq, wv_t, bv, wo_t, bo,
              wb_t, bb):
    def body(h_ref, b_ref, wa_ref, ba_ref, s_ref, wq_ref, bq_ref, wv_ref,
             bv_ref, wo_ref, bo_ref, wb_ref, bb_ref, o_ref):
        gid = lax.broadcasted_iota(jnp.int32, (BLK, GP), 1)
        counts = jnp.zeros((1, GP), F32)
        for i in range(NBLK):
            bat = b_ref[pl.ds(i * BLK, BLK), :]
            mask = (bat == gid).astype(F32)
            counts = counts + jnp.sum(mask, axis=0, keepdims=True)
        r1 = lax.broadcasted_iota(jnp.int32, (GP, GP), 0)
        c1 = lax.broadcasted_iota(jnp.int32, (GP, GP), 1)
        ut = (r1 < c1).astype(F32)
        starts = _dotx(counts, ut)
        starts_col = starts.reshape(GP, 1)
        row0 = lax.broadcasted_iota(jnp.int32, (BLK, 1), 0).astype(F32)
        gsum = jnp.zeros((GP, D), F32)
        for i in range(NBLK):
            bat = b_ref[pl.ds(i * BLK, BLK), :]
            mask = (bat == gid).astype(F32)
            out_b = _dot(h_ref[pl.ds(i * BLK, BLK), :], wa_ref[...]) \
                + ba_ref[...]
            starts_pn = _dotx(mask, starts_col)
            pos = row0 + float(i * BLK) - starts_pn
            validf = jnp.where(pos < float(MAXN), 1.0, 0.0)
            gsum = gsum + lax.dot_general(
                mask, _rd(out_b) * validf, (((0,), (0,)), ((), ())),
                precision=_PH, preferred_element_type=F32)
        q = _dot(s_ref[...], wq_ref[...]) + bq_ref[...]
        o = q + _dotx(gsum, _rd(wv_ref[...])) \
            + float(MAXN) * bv_ref[...]
        o = o + jnp.maximum(_dot(o, wo_ref[...]) + bo_ref[...], 0.0)
        o_ref[...] = _dot(o, wb_ref[...]) + bb_ref[...]
    return pl.pallas_call(
        body, out_shape=jax.ShapeDtypeStruct((GP, 1), F32))(
            h, batchp, wa_t, ba, s_row, wq_t, bq, wv_t, bv, wo_t, bo,
            wb_t, bb)


def kernel(x, edge_index, batch, params):
    p = params
    pr = jnp.arange(EP - E, dtype=jnp.int32)
    src = jnp.concatenate([edge_index[0].astype(jnp.int32), pr % N])
    dst = jnp.concatenate([edge_index[1].astype(jnp.int32),
                           N + pr % (NP - N)])
    sd = jnp.stack([src.reshape(NCHUNK, CH), dst.reshape(NCHUNK, CH)],
                   axis=1)
    xp = jnp.pad(x.astype(F32), ((0, NP - N), (0, 0)))
    batchp = jnp.pad(batch.astype(jnp.int32), (0, NP - N),
                     constant_values=GP - 1).reshape(NP, 1)
    zrows = jnp.zeros((CH, D), F32)
    zdeg = jnp.zeros((CH,), F32)
    ones_ch = jnp.ones((CH,), F32)

    h = _tc_input(xp, p['W_in'].T, p['b_in'].reshape(1, D))

    aggp, degp = _sc_agg_deg(h, sd, zrows, zdeg, ones_ch)
    degp3 = degp.reshape(NC, NP, 1)
    h = _tc_layer(aggp, degp3, h, p['W_l0'].T, p['b_l0'].reshape(1, D),
                  p['W_r0'].T)
    for l in (1, 2):
        aggp = _sc_agg(h, sd, zrows, zdeg, ones_ch)[0]
        h = _tc_layer(aggp, degp3, h, p['W_l%d' % l].T,
                      p['b_l%d' % l].reshape(1, D), p['W_r%d' % l].T)

    res = _tc_final(h, batchp, p['W_a'].T, p['b_a'].reshape(1, D),
                    p['S'].reshape(1, D), p['W_q'].T, p['b_q'].reshape(1, D),
                    p['W_v'].T, p['b_v'].reshape(1, D), p['W_o'].T,
                    p['b_o'].reshape(1, D), p['W_b'].T,
                    p['b_b'].reshape(1, 1))
    return res[:G, 0]


# speedup vs baseline: 3.8272x; 1.0152x over previous
import functools

import jax
import jax.numpy as jnp
from jax import lax
from jax.experimental import pallas as pl
from jax.experimental.pallas import tpu as pltpu
from jax.experimental.pallas import tpu_sc as plsc

N = 10000
E = 320000
D = 128
G = 20
GP = 32
MAXN = 500
NP = 10240
NC = 2
NS = 16
NW = NC * NS
CH = 128
EP = 327680
NCHUNK = EP // CH
CPW = EP // NW // CH
RPT = NP // NS
F32 = jnp.float32
_PH = jax.lax.Precision.HIGHEST

NBUF = 2
LA = 1
IB = 4
LI = 2


def _dot(a, b):
    return jnp.dot(a, b, preferred_element_type=F32)


def _dotx(a, b):
    return jnp.dot(a, b, precision=_PH, preferred_element_type=F32)


def _rd(a):
    return a.astype(jnp.bfloat16).astype(F32)


def _make_sc_agg(compute_deg: bool):
    mesh = plsc.VectorSubcoreMesh(core_axis_name="c", subcore_axis_name="s",
                                  num_cores=NC, num_subcores=NS)
    out_type = [jax.ShapeDtypeStruct((NC, NP, D), F32)]
    if compute_deg:
        out_type.append(jax.ShapeDtypeStruct((NC, NP), F32))
    scratch = [pltpu.VMEM((2, CH), jnp.int32) for _ in range(IB)]
    scratch += [pltpu.VMEM((CH, D), F32) for _ in range(NBUF)]
    scratch += [
        pltpu.VMEM((CH,), F32),
        pltpu.VMEM((CH,), F32),
    ]
    scratch += [pltpu.SemaphoreType.DMA] * (IB + 3 * NBUF)
    scratch += [
        pltpu.VMEM_SHARED((NP, D), F32),
        pltpu.VMEM_SHARED((NP,), F32),
    ]

    def body(h_hbm, sd_hbm, zrows_hbm, zdeg_hbm, ones_hbm, *refs):
        if compute_deg:
            agg_out, deg_out = refs[0], refs[1]
            rest = refs[2:]
        else:
            agg_out = refs[0]
            deg_out = None
            rest = refs[1:]
        islots = list(rest[0:IB])
        bufs = list(rest[IB:IB + NBUF])
        ones_v, zv = rest[IB + NBUF], rest[IB + NBUF + 1]
        sems = list(rest[IB + NBUF + 2:-2])
        agg_sh, deg_sh = rest[-2], rest[-1]
        isem = sems[0:IB]
        gsem = sems[IB:IB + NBUF]
        ssem = sems[IB + NBUF:IB + 2 * NBUF]
        dsem = sems[IB + 2 * NBUF:IB + 3 * NBUF]
        c = lax.axis_index("c")
        s = lax.axis_index("s")
        wid = s * NC + c
        base = wid * CPW

        NZ = RPT // CH
        pltpu.sync_copy(zrows_hbm, bufs[0])
        pltpu.sync_copy(zdeg_hbm, zv)
        for k in range(NZ):
            pltpu.async_copy(bufs[0], agg_sh.at[pl.ds(s * RPT + k * CH, CH)],
                             ssem[k % NBUF])
        if compute_deg:
            for k in range(NZ):
                pltpu.async_copy(zv,
                                 deg_sh.at[pl.ds(s * RPT + k * CH, CH)],
                                 dsem[k % NBUF])
            pltpu.sync_copy(ones_hbm, ones_v)
        for k in range(NZ):
            pltpu.make_async_copy(
                bufs[0], agg_sh.at[pl.ds(s * RPT + k * CH, CH)],
                ssem[k % NBUF]).wait()
            if compute_deg:
                pltpu.make_async_copy(
                    zv, deg_sh.at[pl.ds(s * RPT + k * CH, CH)],
                    dsem[k % NBUF]).wait()
        plsc.subcore_barrier()

        def idx_load(u):
            pltpu.async_copy(sd_hbm.at[base + u], islots[u % IB],
                             isem[u % IB])

        def gather(t, b):
            return pltpu.async_copy(h_hbm.at[islots[t % IB].at[0]], bufs[b],
                                    gsem[b])

        for k in range(LI):
            idx_load(k)
        for k in range(LA):
            pltpu.make_async_copy(sd_hbm.at[base + k], islots[k % IB],
                                  isem[k % IB]).wait()
            gather(k, k % NBUF)

        for t in range(CPW):
            b = t % NBUF
            ti = t + LI
            if ti < CPW:
                idx_load(ti)
            tf = t + LA
            if tf < CPW:
                bf = tf % NBUF
                if tf >= NBUF:
                    pltpu.make_async_copy(
                        bufs[bf], agg_sh.at[islots[(tf - NBUF) % IB].at[1]],
                        ssem[bf]).wait()
                pltpu.make_async_copy(sd_hbm.at[base + tf],
                                      islots[tf % IB], isem[tf % IB]).wait()
                gather(tf, bf)
            pltpu.make_async_copy(h_hbm.at[islots[t % IB].at[0]], bufs[b],
                                  gsem[b]).wait()
            pltpu.async_copy(bufs[b], agg_sh.at[islots[t % IB].at[1]],
                             ssem[b], add=True)
            if compute_deg:
                if t >= NBUF:
                    pltpu.make_async_copy(
                        ones_v, deg_sh.at[islots[(t - NBUF) % IB].at[1]],
                        dsem[b]).wait()
                pltpu.async_copy(ones_v, deg_sh.at[islots[t % IB].at[1]],
                                 dsem[b], add=True)

        for t in range(CPW - NBUF, CPW):
            b = t % NBUF
            pltpu.make_async_copy(bufs[b], agg_sh.at[islots[t % IB].at[1]],
                                  ssem[b]).wait()
            if compute_deg:
                pltpu.make_async_copy(ones_v,
                                      deg_sh.at[islots[t % IB].at[1]],
                                      dsem[b]).wait()
        plsc.subcore_barrier()

        def wb_a(k):
            off = s * RPT + k * CH
            return pltpu.make_async_copy(agg_sh.at[pl.ds(off, CH)],
                                         bufs[k % NBUF], gsem[k % NBUF])

        def wb_b(k):
            off = s * RPT + k * CH
            return pltpu.make_async_copy(bufs[k % NBUF],
                                         agg_out.at[c, pl.ds(off, CH)],
                                         ssem[k % NBUF])

        for k in range(min(NBUF, NZ)):
            wb_a(k).start()
        for k in range(NZ):
            wb_a(k).wait()
            wb_b(k).start()
            if k + NBUF < NZ:
                wb_b(k).wait()
                wb_a(k + NBUF).start()
        for k in range(max(0, NZ - NBUF), NZ):
            wb_b(k).wait()
        if compute_deg:
            dbufs = [zv, ones_v]

            def dg_a(k):
                off = s * RPT + k * CH
                return pltpu.make_async_copy(deg_sh.at[pl.ds(off, CH)],
                                             dbufs[k % 2], dsem[k % 2])

            def dg_b(k):
                off = s * RPT + k * CH
                return pltpu.make_async_copy(dbufs[k % 2],
                                             deg_out.at[c, pl.ds(off, CH)],
                                             isem[k % 2])

            for k in range(min(2, NZ)):
                dg_a(k).start()
            for k in range(NZ):
                dg_a(k).wait()
                dg_b(k).start()
                if k + 2 < NZ:
                    dg_b(k).wait()
                    dg_a(k + 2).start()
            for k in range(max(0, NZ - 2), NZ):
                dg_b(k).wait()

    return pl.kernel(body, out_type=tuple(out_type), mesh=mesh,
                     scratch_types=scratch)


_sc_agg_deg = _make_sc_agg(True)
_sc_agg = _make_sc_agg(False)


BLK = 1024
NBLK = NP // BLK
INF = 16


def _tc_input(xp, w_t, b):
    def body(x_ref, w_ref, b_ref, o_ref):
        o_ref[...] = jnp.maximum(_dot(x_ref[...], w_ref[...]) + b_ref[...],
                                 0.0)
    return pl.pallas_call(
        body,
        grid=(NBLK,),
        in_specs=[
            pl.BlockSpec((BLK, INF), lambda i: (i, 0)),
            pl.BlockSpec((INF, D), lambda i: (0, 0)),
            pl.BlockSpec((1, D), lambda i: (0, 0)),
        ],
        out_specs=pl.BlockSpec((BLK, D), lambda i: (i, 0)),
        out_shape=jax.ShapeDtypeStruct((NP, D), F32))(xp, w_t, b)


def _tc_layer(aggp, degp3, h, wl_t, bl, wr_t):
    def body(a_ref, dg_ref, h_ref, wl_ref, bl_ref, wr_ref, o_ref):
        agg = a_ref[0] + a_ref[1]
        deg = dg_ref[0] + dg_ref[1]
        deginv = 1.0 / jnp.maximum(deg, 1.0)
        m = _dot(agg * deginv, wl_ref[...])
        r = _dot(h_ref[...], wr_ref[...])
        o_ref[...] = jnp.maximum(m + bl_ref[...] + r, 0.0)
    return pl.pallas_call(
        body,
        grid=(NBLK,),
        in_specs=[
            pl.BlockSpec((NC, BLK, D), lambda i: (0, i, 0)),
            pl.BlockSpec((NC, BLK, 1), lambda i: (0, i, 0)),
            pl.BlockSpec((BLK, D), lambda i: (i, 0)),
            pl.BlockSpec((D, D), lambda i: (0, 0)),
            pl.BlockSpec((1, D), lambda i: (0, 0)),
            pl.BlockSpec((D, D), lambda i: (0, 0)),
        ],
        out_specs=pl.BlockSpec((BLK, D), lambda i: (i, 0)),
        out_shape=jax.ShapeDtypeStruct((NP, D), F32))(
            aggp, degp3, h, wl_t, bl, wr_t)


def _tc_final(h, batchp, wa_t, ba, s_row, wq_t, bq, wv_t, bv, wo_t, bo,
              wb_t, bb):
    def body(h_ref, b_ref, wa_ref, ba_ref, s_ref, wq_ref, bq_ref, wv_ref,
             bv_ref, wo_ref, bo_ref, wb_ref, bb_ref, o_ref):
        gid = lax.broadcasted_iota(jnp.int32, (BLK, GP), 1)
        counts = jnp.zeros((1, GP), F32)
        for i in range(NBLK):
            bat = b_ref[pl.ds(i * BLK, BLK), :]
            mask = (bat == gid).astype(F32)
            counts = counts + jnp.sum(mask, axis=0, keepdims=True)
        r1 = lax.broadcasted_iota(jnp.int32, (GP, GP), 0)
        c1 = lax.broadcasted_iota(jnp.int32, (GP, GP), 1)
        ut = (r1 < c1).astype(F32)
        starts = _dotx(counts, ut)
        starts_col = starts.reshape(GP, 1)
        row0 = lax.broadcasted_iota(jnp.int32, (BLK, 1), 0).astype(F32)
        gsum = jnp.zeros((GP, D), F32)
        for i in range(NBLK):
            bat = b_ref[pl.ds(i * BLK, BLK), :]
            mask = (bat == gid).astype(F32)
            out_b = _dot(h_ref[pl.ds(i * BLK, BLK), :], wa_ref[...]) \
                + ba_ref[...]
            starts_pn = _dotx(mask, starts_col)
            pos = row0 + float(i * BLK) - starts_pn
            validf = jnp.where(pos < float(MAXN), 1.0, 0.0)
            gsum = gsum + lax.dot_general(
                mask, _rd(out_b) * validf, (((0,), (0,)), ((), ())),
                precision=_PH, preferred_element_type=F32)
        q = _dot(s_ref[...], wq_ref[...]) + bq_ref[...]
        o = q + _dotx(gsum, _rd(wv_ref[...])) \
            + float(MAXN) * bv_ref[...]
        o = o + jnp.maximum(_dot(o, wo_ref[...]) + bo_ref[...], 0.0)
        o_ref[...] = _dot(o, wb_ref[...]) + bb_ref[...]
    return pl.pallas_call(
        body, out_shape=jax.ShapeDtypeStruct((GP, 1), F32))(
            h, batchp, wa_t, ba, s_row, wq_t, bq, wv_t, bv, wo_t, bo,
            wb_t, bb)


def kernel(x, edge_index, batch, params):
    p = params
    pr = jnp.arange(EP - E, dtype=jnp.int32)
    src = jnp.concatenate([edge_index[0].astype(jnp.int32), pr % N])
    dst = jnp.concatenate([edge_index[1].astype(jnp.int32),
                           N + pr % (NP - N)])
    sd = jnp.stack([src.reshape(NCHUNK, CH), dst.reshape(NCHUNK, CH)],
                   axis=1)
    xp = jnp.pad(x.astype(F32), ((0, NP - N), (0, 0)))
    batchp = jnp.pad(batch.astype(jnp.int32), (0, NP - N),
                     constant_values=GP - 1).reshape(NP, 1)
    zrows = jnp.zeros((CH, D), F32)
    zdeg = jnp.zeros((CH,), F32)
    ones_ch = jnp.ones((CH,), F32)

    h = _tc_input(xp, p['W_in'].T, p['b_in'].reshape(1, D))

    aggp, degp = _sc_agg_deg(h, sd, zrows, zdeg, ones_ch)
    degp3 = degp.reshape(NC, NP, 1)
    h = _tc_layer(aggp, degp3, h, p['W_l0'].T, p['b_l0'].reshape(1, D),
                  p['W_r0'].T)
    for l in (1, 2):
        aggp = _sc_agg(h, sd, zrows, zdeg, ones_ch)[0]
        h = _tc_layer(aggp, degp3, h, p['W_l%d' % l].T,
                      p['b_l%d' % l].reshape(1, D), p['W_r%d' % l].T)

    res = _tc_final(h, batchp, p['W_a'].T, p['b_a'].reshape(1, D),
                    p['S'].reshape(1, D), p['W_q'].T, p['b_q'].reshape(1, D),
                    p['W_v'].T, p['b_v'].reshape(1, D), p['W_o'].T,
                    p['b_o'].reshape(1, D), p['W_b'].T,
                    p['b_b'].reshape(1, 1))
    return res[:G, 0]
